# Initial kernel scaffold; baseline (speedup 1.0000x reference)
import jax
import jax.numpy as jnp
from jax.experimental import pallas as pl


def kernel(feats, coords, W1, W2, W3, g1, b1, g2, b2, g3, b3):
    raise NotImplementedError("write your pallas kernel here")


# speedup vs baseline: 3.5731x; 3.5731x over previous
import functools

import jax
import jax.numpy as jnp
from jax import lax
from jax.experimental import pallas as pl
from jax.experimental.pallas import tpu as pltpu
from jax.experimental.pallas import tpu_sc as plsc

D0, D1, D2 = 480, 360, 32
D12 = D1 * D2
TBL = D0 * D1 * D2
PADKEY = TBL + 16
TBL_P = TBL + 32
C = 32

NC, NS = 2, 16
NW = NC * NS
SUB = 128
NSUB = 5
CH = SUB * NSUB
NCHUNK = 10
RPT = CH * NCHUNK
NP = RPT * NW

TAP_OFF = (-D12, D12, -D2, D2, -1, 1)
TAP_AXIS = (0, 0, 1, 1, 2, 2)
TAP_D = (-1, 1, -1, 1, -1, 1)
DIMS = (D0, D1, D2)

_mesh = plsc.VectorSubcoreMesh(core_axis_name="c", subcore_axis_name="s")


def _wid():
    return lax.axis_index("s") * NC + lax.axis_index("c")


@functools.partial(
    pl.kernel,
    out_type=[
        jax.ShapeDtypeStruct((TBL_P,), jnp.int32),
        jax.ShapeDtypeStruct((NP,), jnp.int32),
    ],
    mesh=_mesh,
    scratch_types=[
        pltpu.VMEM((CH,), jnp.int32),
        pltpu.VMEM((CH,), jnp.int32),
        pltpu.VMEM((CH,), jnp.int32),
        pltpu.VMEM((NSUB, SUB), jnp.int32),
        pltpu.VMEM((NSUB, SUB), jnp.int32),
        pltpu.SemaphoreType.DMA,
    ],
)
def _sc_build_table(c0h, c1h, c2h, tableh, keysh, c0b, c1b, c2b, kb, vb, sem):
    base = _wid() * RPT
    iota = lax.iota(jnp.int32, 16)

    def chunk(ch, _):
        rb = base + ch * CH
        pltpu.sync_copy(c0h.at[pl.ds(rb, CH)], c0b)
        pltpu.sync_copy(c1h.at[pl.ds(rb, CH)], c1b)
        pltpu.sync_copy(c2h.at[pl.ds(rb, CH)], c2b)
        for s in range(NSUB):
            def body(i, _):
                o = s * SUB + i * 16
                key = c0b[pl.ds(o, 16)] * D12 + c1b[pl.ds(o, 16)] * D2 + c2b[pl.ds(o, 16)]
                kb[s, pl.ds(i * 16, 16)] = key
                vb[s, pl.ds(i * 16, 16)] = rb + o + iota
                return _
            lax.fori_loop(0, SUB // 16, body, None)
        for s in range(NSUB):
            pltpu.sync_copy(kb.at[s], keysh.at[pl.ds(rb + s * SUB, SUB)])
        hs = [pltpu.async_copy(vb.at[s], tableh.at[kb.at[s]], sem)
              for s in range(NSUB)]
        for h in hs:
            h.wait()
        return _

    lax.fori_loop(0, NCHUNK, chunk, None)


@functools.partial(
    pl.kernel,
    out_type=jax.ShapeDtypeStruct((6, NP, C), jnp.float32),
    mesh=_mesh,
    compiler_params=pltpu.CompilerParams(use_tc_tiling_on_sc=False),
    scratch_types=[
        pltpu.VMEM((CH,), jnp.int32),
        pltpu.VMEM((CH,), jnp.int32),
        pltpu.VMEM((6, NSUB, SUB), jnp.int32),
        pltpu.VMEM((6, NSUB, SUB), jnp.int32),
        pltpu.VMEM((6, NSUB, SUB), jnp.int32),
        pltpu.VMEM((6, NSUB, SUB), jnp.int32),
        pltpu.VMEM((6, NSUB, SUB), jnp.int32),
        pltpu.VMEM((6, NSUB, SUB), jnp.int32),
        pltpu.VMEM((2, CH, C), jnp.float32),
        pltpu.SemaphoreType.DMA,
        pltpu.SemaphoreType.DMA,
        pltpu.SemaphoreType.DMA,
        pltpu.SemaphoreType.DMA,
    ],
)
def _sc_gather(c0h, c1h, c2h, keysh, tableh, fpadh, gh,
               cb, keyb, slotb, nkeb, jb, jcb, kvb, fjb, rowsb,
               semj, semk, semr0, semr1):
    base = _wid() * RPT
    n_real = jnp.int32(200000)

    def chunk(ch, _):
        rb = base + ch * CH
        pltpu.sync_copy(keysh.at[pl.ds(rb, CH)], keyb)
        for t in range(6):
            ca_h = (c0h, c1h, c2h)[TAP_AXIS[t]]
            pltpu.sync_copy(ca_h.at[pl.ds(rb, CH)], cb)
            off = TAP_OFF[t]
            d = TAP_D[t]
            dim = DIMS[TAP_AXIS[t]]
            for s in range(NSUB):
                def body(i, _):
                    o = s * SUB + i * 16
                    ca = cb[pl.ds(o, 16)] + d
                    inb = (ca >= 0) & (ca < dim)
                    nk = keyb[pl.ds(o, 16)] + off
                    slotb[t, s, pl.ds(i * 16, 16)] = jnp.where(inb, nk, 0)
                    nkeb[t, s, pl.ds(i * 16, 16)] = jnp.where(inb, nk, -7)
                    return _
                lax.fori_loop(0, SUB // 16, body, None)
        hs = [pltpu.async_copy(tableh.at[slotb.at[t, s]], jb.at[t, s], semj)
              for t in range(6) for s in range(NSUB)]
        for h in hs:
            h.wait()
        for t in range(6):
            for s in range(NSUB):
                def body(i, _):
                    sl = pl.ds(i * 16, 16)
                    jcb[t, s, sl] = jnp.minimum(jnp.maximum(jb[t, s, sl], 0), n_real)
                    return _
                lax.fori_loop(0, SUB // 16, body, None)
        hs = [pltpu.async_copy(keysh.at[jcb.at[t, s]], kvb.at[t, s], semk)
              for t in range(6) for s in range(NSUB)]
        for h in hs:
            h.wait()
        for t in range(6):
            for s in range(NSUB):
                def body(i, _):
                    sl = pl.ds(i * 16, 16)
                    ok = kvb[t, s, sl] == nkeb[t, s, sl]
                    fjb[t, s, sl] = jnp.where(ok, jcb[t, s, sl], n_real)
                    return _
                lax.fori_loop(0, SUB // 16, body, None)
        hprev = None
        for t in range(6):
            db = t % 2
            hs = [pltpu.async_copy(fpadh.at[fjb.at[t, s]],
                                   rowsb.at[db, pl.ds(s * SUB, SUB), :],
                                   (semr0, semr1)[db])
                  for s in range(NSUB)]
            if hprev is not None:
                for h in hprev:
                    h.wait()
                pltpu.sync_copy(rowsb.at[1 - db], gh.at[t - 1, pl.ds(rb, CH), :])
            hprev = hs
        for h in hprev:
            h.wait()
        pltpu.sync_copy(rowsb.at[1], gh.at[5, pl.ds(rb, CH), :])
        return _

    lax.fori_loop(0, NCHUNK, chunk, None)


BLK = 2000
NBLK = 100


def _tc_conv_body(f_ref, g_ref, w_ref, out_ref, sum_ref, sq_ref, acc_s, acc_q):
    i = pl.program_id(0)
    x = f_ref[...]
    g = g_ref[...]
    xcat = jnp.concatenate([g[0], g[1], g[2], g[3], g[4], g[5], x], axis=1)
    o = jnp.dot(xcat, w_ref[...], preferred_element_type=jnp.float32)
    out_ref[...] = o
    s = jnp.broadcast_to(jnp.sum(o, axis=0, keepdims=True), (8, 96))
    q = jnp.broadcast_to(jnp.sum(o * o, axis=0, keepdims=True), (8, 96))

    @pl.when(i == 0)
    def _():
        acc_s[...] = s
        acc_q[...] = q

    @pl.when(i > 0)
    def _():
        acc_s[...] += s
        acc_q[...] += q

    @pl.when(i == NBLK - 1)
    def _():
        sum_ref[...] = acc_s[...]
        sq_ref[...] = acc_q[...]


def _tc_final_body(o_ref, f_ref, sum_ref, sq_ref, g_ref, b_ref, out_ref):
    n = jnp.float32(200000.0)
    m = sum_ref[0:1, :] / n
    v = sq_ref[0:1, :] / n - m * m
    inv = lax.rsqrt(v + 1e-5)
    z = (o_ref[...] - m) * inv * g_ref[0:1, :] + b_ref[0:1, :]
    y = 1.0 / (1.0 + jnp.exp(-z))
    out_ref[...] = (y[:, 0:32] + y[:, 32:64] + y[:, 64:96]) * f_ref[...]


def kernel(feats, coords, W1, W2, W3, g1, b1, g2, b2, g3, b3):
    n = feats.shape[0]
    npad = NP - n
    ct = coords.T.astype(jnp.int32)
    padc = jnp.tile(jnp.array([[D0], [0], [16]], jnp.int32), (1, npad))
    ct = jnp.concatenate([ct, padc], axis=1)
    c0, c1, c2 = ct[0], ct[1], ct[2]
    fpad = jnp.concatenate([feats, jnp.zeros((8, C), jnp.float32)], axis=0)

    Z = jnp.zeros((C, C), jnp.float32)
    rows = [
        jnp.concatenate([W1[0], Z, Z], 1),
        jnp.concatenate([W1[2], Z, Z], 1),
        jnp.concatenate([Z, W2[0], Z], 1),
        jnp.concatenate([Z, W2[2], Z], 1),
        jnp.concatenate([Z, Z, W3[0]], 1),
        jnp.concatenate([Z, Z, W3[2]], 1),
        jnp.concatenate([W1[1], W2[1], W3[1]], 1),
    ]
    wbig = jnp.concatenate(rows, axis=0)
    gcat = jnp.broadcast_to(jnp.concatenate([g1, g2, g3])[None, :], (8, 96))
    bcat = jnp.broadcast_to(jnp.concatenate([b1, b2, b3])[None, :], (8, 96))

    table, keys = _sc_build_table(c0, c1, c2)
    g6 = _sc_gather(c0, c1, c2, keys, table, fpad)

    out96, sums, sqs = pl.pallas_call(
        _tc_conv_body,
        grid=(NBLK,),
        in_specs=[
            pl.BlockSpec((BLK, C), lambda i: (i, 0)),
            pl.BlockSpec((6, BLK, C), lambda i: (0, i, 0)),
            pl.BlockSpec((224, 96), lambda i: (0, 0)),
        ],
        out_specs=[
            pl.BlockSpec((BLK, 96), lambda i: (i, 0)),
            pl.BlockSpec((8, 96), lambda i: (0, 0)),
            pl.BlockSpec((8, 96), lambda i: (0, 0)),
        ],
        out_shape=[
            jax.ShapeDtypeStruct((n, 96), jnp.float32),
            jax.ShapeDtypeStruct((8, 96), jnp.float32),
            jax.ShapeDtypeStruct((8, 96), jnp.float32),
        ],
        scratch_shapes=[
            pltpu.VMEM((8, 96), jnp.float32),
            pltpu.VMEM((8, 96), jnp.float32),
        ],
    )(feats, g6, wbig)

    out = pl.pallas_call(
        _tc_final_body,
        grid=(NBLK,),
        in_specs=[
            pl.BlockSpec((BLK, 96), lambda i: (i, 0)),
            pl.BlockSpec((BLK, C), lambda i: (i, 0)),
            pl.BlockSpec((8, 96), lambda i: (0, 0)),
            pl.BlockSpec((8, 96), lambda i: (0, 0)),
            pl.BlockSpec((8, 96), lambda i: (0, 0)),
            pl.BlockSpec((8, 96), lambda i: (0, 0)),
        ],
        out_specs=pl.BlockSpec((BLK, C), lambda i: (i, 0)),
        out_shape=jax.ShapeDtypeStruct((n, C), jnp.float32),
    )(out96, feats, sums, sqs, gcat, bcat)
    return out
